# baseline (speedup 1.0000x reference)
import jax
import jax.numpy as jnp
from jax.experimental import pallas as pl
from jax.experimental.pallas import tpu as pltpu

B, N, D = 32, 77, 768
HEADS, DH = 8, 64
INNER = HEADS * DH
FF = 4 * D
V = 10
OUT = V * D


def _ln(t, g, b):
    m = jnp.mean(t, axis=-1, keepdims=True)
    var = jnp.mean((t - m) ** 2, axis=-1, keepdims=True)
    return (t - m) / jnp.sqrt(var + 1e-5) * g + b


def _u_kernel(x_ref, g1_ref, b1_ref, wv1_ref, wo1_ref, bo1_ref,
              wv2_ref, wo2_ref, bo2_ref, g3_ref, b3_ref,
              w1_ref, b1f_ref, u_ref):
    x = x_ref[...]
    nx = _ln(x, g1_ref[...], b1_ref[...])
    h = jnp.dot(jnp.dot(nx, wv1_ref[...], preferred_element_type=jnp.float32),
                wo1_ref[...], preferred_element_type=jnp.float32) + bo1_ref[...] + x
    h = jnp.dot(jnp.dot(x, wv2_ref[...], preferred_element_type=jnp.float32),
                wo2_ref[...], preferred_element_type=jnp.float32) + bo2_ref[...] + h
    n3 = _ln(h, g3_ref[...], b3_ref[...])
    t = jnp.dot(n3, w1_ref[...], preferred_element_type=jnp.float32) + b1f_ref[...]
    a = t[:, :FF]
    gate = t[:, FF:]
    u_ref[...] = a * jax.nn.gelu(gate)


def _pe_kernel(u_ref, w2_ref, b2_ref, pe_ref):
    pe_ref[0] = jnp.dot(u_ref[...], w2_ref[...],
                        preferred_element_type=jnp.float32) + b2_ref[0]


def _select_kernel(ph_ref, tok_ref, emb_ref, pe_ref, out_ref):
    mask = tok_ref[...] == ph_ref[0]
    out_ref[0] = jnp.where(mask, pe_ref[0], emb_ref[...])


def kernel(tokenized_text, embedded_text, initial_embeddings, params, placeholder_token):
    x = initial_embeddings[-1:].reshape(1, D)
    p1, p2, ff = params['attn1'], params['attn2'], params['ff']
    row = lambda a: a.reshape(1, -1)

    u = pl.pallas_call(
        _u_kernel,
        out_shape=jax.ShapeDtypeStruct((1, FF), jnp.float32),
    )(x,
      row(params['ln1']['g']), row(params['ln1']['b']),
      p1['wv'], p1['wo'], row(p1['bo']),
      p2['wv'], p2['wo'], row(p2['bo']),
      row(params['ln3']['g']), row(params['ln3']['b']),
      ff['w1'], row(ff['b1']))

    pe = pl.pallas_call(
        _pe_kernel,
        grid=(V,),
        in_specs=[
            pl.BlockSpec((1, FF), lambda v: (0, 0)),
            pl.BlockSpec((FF, D), lambda v: (0, v)),
            pl.BlockSpec((1, 1, D), lambda v: (v, 0, 0)),
        ],
        out_specs=pl.BlockSpec((1, 1, D), lambda v: (v, 0, 0)),
        out_shape=jax.ShapeDtypeStruct((V, 1, D), jnp.float32),
    )(u, ff['w2'], ff['b2'].reshape(V, 1, D))

    tok = tokenized_text.reshape(B, N, 1).astype(jnp.int32)
    ph = jnp.asarray(placeholder_token, jnp.int32).reshape(1)

    outs = pl.pallas_call(
        _select_kernel,
        grid=(V,),
        in_specs=[
            pl.BlockSpec(memory_space=pltpu.SMEM),
            pl.BlockSpec((B, N, 1), lambda v: (0, 0, 0)),
            pl.BlockSpec((B, N, D), lambda v: (0, 0, 0)),
            pl.BlockSpec((1, 1, D), lambda v: (v, 0, 0)),
        ],
        out_specs=pl.BlockSpec((1, B, N, D), lambda v: (v, 0, 0, 0)),
        out_shape=jax.ShapeDtypeStruct((V, B, N, D), jnp.float32),
    )(ph, tok, embedded_text, pe)

    return outs
